# baseline (speedup 1.0000x reference)
import functools

import jax
import jax.numpy as jnp
from jax import lax
from jax.experimental import pallas as pl
from jax.experimental.pallas import tpu as pltpu
from jax.experimental.pallas import tpu_sc as plsc

N = 25000
C = 128
E = 300000
NSEED = 512
NPAD = 25088
EPAD = 307200
HALF = 12544
ACC_BIG = 12560
ACC_SML = 528
ROWBLK = 32
NSLOT = 4
BIGVAL = 1 << 28


def _mesh():
    return plsc.VectorSubcoreMesh(core_axis_name="c", subcore_axis_name="s")


def _compact_loop(srcv, dstv, srcc, dstc, lo, hi, n_vec, fill0=None):
    def body(i, fill):
        vs = srcv[pl.ds(i * 16, 16)]
        vd = dstv[pl.ds(i * 16, 16)]
        vdl = vd - lo
        m = (vd >= lo) & (vd < hi)
        mi = m.astype(jnp.int32)
        pos = fill + jnp.cumsum(mi) - 1
        plsc.store_scatter(srcc, [pos], vs, mask=m)
        plsc.store_scatter(dstc, [pos], vdl, mask=m)
        cnt = jnp.sum(mi, axis=0)
        return fill + cnt
    if fill0 is None:
        fill0 = jnp.int32(0)
    return lax.fori_loop(0, n_vec, body, fill0)


def _pad_tail(srcc, dstc, fill, dummy):
    zs = jnp.zeros((16,), jnp.int32)
    dd = jnp.full((16,), dummy, jnp.int32)
    for j in range(ROWBLK // 16):
        srcc[pl.ds(fill + j * 16, 16)] = zs
        dstc[pl.ds(fill + j * 16, 16)] = dd


def _gather_blk(h_hbm, srcc, rows4, slot, j, sem):
    return pltpu.make_async_copy(
        h_hbm.at[srcc.at[pl.ds(j * ROWBLK, ROWBLK)]], rows4.at[slot], sem)


def _scatter_start(acc, rows4, d2, slot, sem):
    pltpu.async_copy(rows4.at[slot], acc.at[d2.at[slot]], sem, add=True)


def _scatter_wait(acc, rows4, d2, slot, sem):
    pltpu.make_async_copy(rows4.at[slot], acc.at[d2.at[slot]], sem).wait()


def _flush_blocks(h_hbm, acc, srcc, dstc, rows4, d2, gsems, ssems, nb):
    for p in range(NSLOT - 1):
        @pl.when(p < nb)
        def _(p=p):
            _gather_blk(h_hbm, srcc, rows4, p, p, gsems[p]).start()

    def body(j, _):
        def step(slot):
            _gather_blk(h_hbm, srcc, rows4, slot, j, gsems[slot]).wait()
            for t in range(ROWBLK // 16):
                d2[slot, pl.ds(t * 16, 16)] = \
                    dstc[pl.ds(j * ROWBLK + t * 16, 16)]
            _scatter_start(acc, rows4, d2, slot, ssems[slot])

            nslot = (slot + NSLOT - 1) % NSLOT

            @pl.when(j + NSLOT - 1 < nb)
            def _():
                @pl.when(j >= 1)
                def _():
                    _scatter_wait(acc, rows4, d2, nslot, ssems[nslot])
                _gather_blk(h_hbm, srcc, rows4, nslot, j + NSLOT - 1,
                            gsems[nslot]).start()

        for p in range(NSLOT):
            @pl.when(j % NSLOT == p)
            def _(p=p):
                step(p)

        return 0

    lax.fori_loop(0, nb, body, 0)
    for p in range(NSLOT):
        jj = nb - 1 - p

        @pl.when(jj >= 0)
        def _(jj=jj):
            for q in range(NSLOT):
                @pl.when(jj % NSLOT == q)
                def _(q=q):
                    _scatter_wait(acc, rows4, d2, q, ssems[q])


def _zero_acc(acc, zb, s, rows_per_tile):
    z = jnp.zeros((16,), jnp.float32)
    for i in range(16):
        for j in range(8):
            zb[i, pl.ds(j * 16, 16)] = z
    row0 = s * rows_per_tile

    def body(k, _):
        pltpu.sync_copy(zb, acc.at[pl.ds(row0 + k * 16, 16)])
        return 0

    lax.fori_loop(0, rows_per_tile // 16, body, 0)


_EPT_BIG = EPAD // 16
_EPT_SML = EPAD // 32
_CHUNK = 1600
_NCH_BIG = _EPT_BIG // _CHUNK
_NCH_SML = _EPT_SML // _CHUNK


def _seg_phase(h_hbm, src_hbm, dst_hbm, acc, bufs, gsems, ssems, semS,
               e_base, nch, lo, hi, dummy):
    srcv, dstv, srcc, dstc, rows4, d2 = bufs

    def stage(k):
        return (pltpu.make_async_copy(
                    src_hbm.at[pl.ds(e_base + k * _CHUNK, _CHUNK)], srcv, semS),
                pltpu.make_async_copy(
                    dst_hbm.at[pl.ds(e_base + k * _CHUNK, _CHUNK)], dstv, semS))

    for cp in stage(0):
        cp.start()

    def chunk(k, fill):
        for cp in stage(k):
            cp.wait()
        fill = _compact_loop(srcv, dstv, srcc, dstc, lo, hi, _CHUNK // 16, fill)

        @pl.when(k + 1 < nch)
        def _():
            for cp in stage(k + 1):
                cp.start()

        nb = fill // ROWBLK
        _flush_blocks(h_hbm, acc, srcc, dstc, rows4, d2, gsems, ssems, nb)
        base = nb * ROWBLK
        for j in range(ROWBLK // 16):
            vs = srcc[pl.ds(base + j * 16, 16)]
            vd = dstc[pl.ds(base + j * 16, 16)]
            srcc[pl.ds(j * 16, 16)] = vs
            dstc[pl.ds(j * 16, 16)] = vd
        return fill - base

    fill = lax.fori_loop(0, nch, chunk, jnp.int32(0))
    _pad_tail(srcc, dstc, fill, dummy)
    _flush_blocks(h_hbm, acc, srcc, dstc, rows4, d2, gsems, ssems,
                  (fill + (ROWBLK - 1)) // ROWBLK)


@functools.partial(
    pl.kernel,
    out_type=[jax.ShapeDtypeStruct((NPAD, C), jnp.float32),
              jax.ShapeDtypeStruct((2, NSEED, C), jnp.float32)],
    mesh=_mesh(),
    compiler_params=pltpu.CompilerParams(needs_layout_passes=False),
    scratch_types=[
        pltpu.VMEM_SHARED((ACC_BIG, C), jnp.float32),
        pltpu.VMEM_SHARED((ACC_SML, C), jnp.float32),
        pltpu.VMEM((_CHUNK,), jnp.int32),
        pltpu.VMEM((_CHUNK,), jnp.int32),
        pltpu.VMEM((_CHUNK + ROWBLK,), jnp.int32),
        pltpu.VMEM((_CHUNK + ROWBLK,), jnp.int32),
        pltpu.VMEM((NSLOT, ROWBLK, C), jnp.float32),
        pltpu.VMEM((NSLOT, ROWBLK), jnp.int32),
        pltpu.VMEM((16, C), jnp.float32),
        pltpu.SemaphoreType.DMA,
        pltpu.SemaphoreType.DMA,
        pltpu.SemaphoreType.DMA,
        pltpu.SemaphoreType.DMA,
        pltpu.SemaphoreType.DMA,
        pltpu.SemaphoreType.DMA,
        pltpu.SemaphoreType.DMA,
        pltpu.SemaphoreType.DMA,
        pltpu.SemaphoreType.DMA,
    ],
)
def _segsum_fused(ha_hbm, hb_hbm, srcab, dstab, srcba, dstba,
                  out_big, out_sml,
                  acc, acc2, srcv, dstv, srcc, dstc, rows4, d2, zb,
                  g0, g1, g2, g3, s0, s1, s2, s3, semS):
    gsems = [g0, g1, g2, g3]
    ssems = [s0, s1, s2, s3]
    bufs = (srcv, dstv, srcc, dstc, rows4, d2)
    c = lax.axis_index("c")
    s = lax.axis_index("s")
    lo = c * HALF

    _zero_acc(acc, zb, s, HALF // 16)
    _zero_acc(acc2, zb, s, NSEED // 16)

    @pl.when(s == 0)
    def _():
        pltpu.sync_copy(zb, acc.at[pl.ds(HALF, 16)])
        pltpu.sync_copy(zb, acc2.at[pl.ds(NSEED, 16)])

    plsc.subcore_barrier()
    _seg_phase(ha_hbm, srcab, dstab, acc, bufs, gsems, ssems, semS,
               s * _EPT_BIG, _NCH_BIG, lo, lo + HALF, HALF)
    _seg_phase(hb_hbm, srcba, dstba, acc2, bufs, gsems, ssems, semS,
               (c * 16 + s) * _EPT_SML, _NCH_SML,
               jnp.int32(0), jnp.int32(NSEED), NSEED)
    plsc.subcore_barrier()
    per = HALF // 16
    pltpu.sync_copy(acc.at[pl.ds(s * per, per)],
                    out_big.at[pl.ds(c * HALF + s * per, per)])
    per2 = NSEED // 16
    pltpu.sync_copy(acc2.at[pl.ds(s * per2, per2)],
                    out_sml.at[c, pl.ds(s * per2, per2)])


@functools.partial(
    pl.kernel,
    out_type=jax.ShapeDtypeStruct((2, NSEED, C), jnp.float32),
    mesh=_mesh(),
    compiler_params=pltpu.CompilerParams(needs_layout_passes=False),
    scratch_types=[
        pltpu.VMEM_SHARED((ACC_SML, C), jnp.float32),
        pltpu.VMEM((_EPT_SML,), jnp.int32),
        pltpu.VMEM((_EPT_SML,), jnp.int32),
        pltpu.VMEM((_EPT_SML + ROWBLK,), jnp.int32),
        pltpu.VMEM((_EPT_SML + ROWBLK,), jnp.int32),
        pltpu.VMEM((NSLOT, ROWBLK, C), jnp.float32),
        pltpu.VMEM((NSLOT, ROWBLK), jnp.int32),
        pltpu.VMEM((16, C), jnp.float32),
        pltpu.SemaphoreType.DMA,
        pltpu.SemaphoreType.DMA,
        pltpu.SemaphoreType.DMA,
        pltpu.SemaphoreType.DMA,
        pltpu.SemaphoreType.DMA,
        pltpu.SemaphoreType.DMA,
        pltpu.SemaphoreType.DMA,
        pltpu.SemaphoreType.DMA,
    ],
)
def _segsum_seed(h_hbm, src_hbm, dst_hbm, out_hbm,
                 acc, srcv, dstv, srcc, dstc, rows4, d2, zb,
                 g0, g1, g2, g3, s0, s1, s2, s3):
    gsems = [g0, g1, g2, g3]
    ssems = [s0, s1, s2, s3]
    c = lax.axis_index("c")
    s = lax.axis_index("s")
    e0 = (c * 16 + s) * _EPT_SML
    pltpu.sync_copy(src_hbm.at[pl.ds(e0, _EPT_SML)], srcv)
    pltpu.sync_copy(dst_hbm.at[pl.ds(e0, _EPT_SML)], dstv)
    _zero_acc(acc, zb, s, NSEED // 16)

    @pl.when(s == 0)
    def _():
        pltpu.sync_copy(zb, acc.at[pl.ds(NSEED, 16)])

    plsc.subcore_barrier()
    fill = _compact_loop(srcv, dstv, srcc, dstc, jnp.int32(0), jnp.int32(NSEED),
                         _EPT_SML // 16)
    _pad_tail(srcc, dstc, fill, NSEED)
    _flush_blocks(h_hbm, acc, srcc, dstc, rows4, d2, gsems, ssems,
                  (fill + (ROWBLK - 1)) // ROWBLK)
    plsc.subcore_barrier()
    per = NSEED // 16
    pltpu.sync_copy(acc.at[pl.ds(s * per, per)],
                    out_hbm.at[c, pl.ds(s * per, per)])


_RB = 256


def _enc2_body(xa_ref, ba_ref, ta_ref, xb_ref, bb_ref, tb_ref, seed_ref,
               Wa_ref, bia_ref, Wta_ref, Wb_ref, bib_ref, Wtb_ref,
               oa_ref, ob_ref):
    seed_row = seed_ref[...]
    cols = lax.broadcasted_iota(jnp.int32, (_RB, NSEED), 1)

    def enc(x_ref, b2_ref, t2_ref, W_ref, bias_ref, Wt_ref, o_ref):
        oh = (b2_ref[...] == cols).astype(jnp.float32)
        st = jnp.sum(oh * seed_row, axis=1, keepdims=True)
        dt = st - t2_ref[...]
        h = jnp.dot(x_ref[...], W_ref[...], preferred_element_type=jnp.float32)
        o_ref[...] = h + bias_ref[...] + dt * Wt_ref[...]

    enc(xa_ref, ba_ref, ta_ref, Wa_ref, bia_ref, Wta_ref, oa_ref)
    enc(xb_ref, bb_ref, tb_ref, Wb_ref, bib_ref, Wtb_ref, ob_ref)


def _encode2(xa, ba2, ta2, xb, bb2, tb2, seed_row,
             Wa, bias_a, Wta, Wb, bias_b, Wtb):
    grid = (NPAD // _RB,)
    row = pl.BlockSpec((_RB, C), lambda i: (i, 0))
    one = pl.BlockSpec((_RB, 1), lambda i: (i, 0))
    wspec = pl.BlockSpec((C, C), lambda i: (0, 0))
    vspec = pl.BlockSpec((1, C), lambda i: (0, 0))
    return pl.pallas_call(
        _enc2_body,
        grid=grid,
        in_specs=[row, one, one, row, one, one,
                  pl.BlockSpec((1, NSEED), lambda i: (0, 0)),
                  wspec, vspec, vspec, wspec, vspec, vspec],
        out_specs=[row, row],
        out_shape=[jax.ShapeDtypeStruct((NPAD, C), jnp.float32),
                   jax.ShapeDtypeStruct((NPAD, C), jnp.float32)],
    )(xa, ba2, ta2, xb, bb2, tb2, seed_row,
      Wa, bias_a, Wta, Wb, bias_b, Wtb)


def _combine_body(h_ref, a_ref, Ws_ref, Wn_ref, b_ref, o_ref):
    acc = jnp.dot(h_ref[...], Ws_ref[...], preferred_element_type=jnp.float32)
    acc = acc + jnp.dot(a_ref[...], Wn_ref[...], preferred_element_type=jnp.float32)
    o_ref[...] = jnp.maximum(acc + b_ref[...], 0.0)


def _combine(h, agg, Ws, Wn, b):
    grid = (NPAD // _RB,)
    return pl.pallas_call(
        _combine_body,
        grid=grid,
        in_specs=[
            pl.BlockSpec((_RB, C), lambda i: (i, 0)),
            pl.BlockSpec((_RB, C), lambda i: (i, 0)),
            pl.BlockSpec((C, C), lambda i: (0, 0)),
            pl.BlockSpec((C, C), lambda i: (0, 0)),
            pl.BlockSpec((1, C), lambda i: (0, 0)),
        ],
        out_specs=pl.BlockSpec((_RB, C), lambda i: (i, 0)),
        out_shape=jax.ShapeDtypeStruct((NPAD, C), jnp.float32),
    )(h, agg, Ws, Wn, b)


def _head_body(h_ref, a0_ref, a1_ref, Ws0_ref, Wn0_ref, b0_ref,
               Ws1_ref, Wn1_ref, b1_ref, Wh_ref, bh_ref, o_ref):
    agg0 = a0_ref[0] + a0_ref[1]
    agg1 = a1_ref[0] + a1_ref[1]
    h1 = jnp.dot(h_ref[...], Ws0_ref[...], preferred_element_type=jnp.float32)
    h1 = h1 + jnp.dot(agg0, Wn0_ref[...], preferred_element_type=jnp.float32)
    h1 = jnp.maximum(h1 + b0_ref[...], 0.0)
    h2 = jnp.dot(h1, Ws1_ref[...], preferred_element_type=jnp.float32)
    h2 = h2 + jnp.dot(agg1, Wn1_ref[...], preferred_element_type=jnp.float32)
    h2 = jnp.maximum(h2 + b1_ref[...], 0.0)
    o_ref[...] = jnp.dot(h2, Wh_ref[...], preferred_element_type=jnp.float32) + bh_ref[...]


def _head(h512, a0p, a1p, Ws0, Wn0, b0, Ws1, Wn1, b1, Whp, bhp):
    return pl.pallas_call(
        _head_body,
        grid=(1,),
        in_specs=[
            pl.BlockSpec((NSEED, C), lambda i: (0, 0)),
            pl.BlockSpec((2, NSEED, C), lambda i: (0, 0, 0)),
            pl.BlockSpec((2, NSEED, C), lambda i: (0, 0, 0)),
            pl.BlockSpec((C, C), lambda i: (0, 0)),
            pl.BlockSpec((C, C), lambda i: (0, 0)),
            pl.BlockSpec((1, C), lambda i: (0, 0)),
            pl.BlockSpec((C, C), lambda i: (0, 0)),
            pl.BlockSpec((C, C), lambda i: (0, 0)),
            pl.BlockSpec((1, C), lambda i: (0, 0)),
            pl.BlockSpec((C, C), lambda i: (0, 0)),
            pl.BlockSpec((1, C), lambda i: (0, 0)),
        ],
        out_specs=pl.BlockSpec((NSEED, C), lambda i: (0, 0)),
        out_shape=jax.ShapeDtypeStruct((NSEED, C), jnp.float32),
    )(h512, a0p, a1p, Ws0, Wn0, b0, Ws1, Wn1, b1, Whp, bhp)


def _pad_rows(x, n):
    return jnp.pad(x, ((0, n - x.shape[0]),) + ((0, 0),) * (x.ndim - 1))


def _prep_edges(edge_index):
    src = edge_index[0].astype(jnp.int32)
    dst = edge_index[1].astype(jnp.int32)
    src = jnp.pad(src, (0, EPAD - E))
    dst = jnp.pad(dst, (0, EPAD - E), constant_values=BIGVAL)
    return src, dst


def kernel(x_a, x_b, seed_time, time_a, time_b, edge_index_ab, edge_index_ba,
           batch_a, batch_b, W_enc_a, b_enc_a, W_enc_b, b_enc_b, W_time_a,
           b_time_a, W_time_b, b_time_b, W_self_a, W_neigh_ba, b_gnn_a,
           W_self_b, W_neigh_ab, b_gnn_b, W_head, b_head):
    f32 = jnp.float32
    xa = _pad_rows(x_a, NPAD)
    xb = _pad_rows(x_b, NPAD)
    ba2 = _pad_rows(batch_a.astype(jnp.int32)[:, None], NPAD)
    bb2 = _pad_rows(batch_b.astype(jnp.int32)[:, None], NPAD)
    ta2 = _pad_rows(time_a[:, None], NPAD)
    tb2 = _pad_rows(time_b[:, None], NPAD)
    seed_row = seed_time[None, :]
    src_ab, dst_ab = _prep_edges(edge_index_ab)
    src_ba, dst_ba = _prep_edges(edge_index_ba)

    h_a0, h_b0 = _encode2(xa, ba2, ta2, xb, bb2, tb2, seed_row,
                          W_enc_a, b_enc_a[None, :], W_time_a,
                          W_enc_b, b_enc_b[None, :], W_time_b)

    agg_b0, agg_a0p = _segsum_fused(h_a0, h_b0, src_ab, dst_ab,
                                    src_ba, dst_ba)

    h_b1 = _combine(h_b0, agg_b0, W_self_b[0], W_neigh_ab[0], b_gnn_b[0][None, :])
    agg_a1p = _segsum_seed(h_b1, src_ba, dst_ba)

    Whp = jnp.pad(W_head, ((0, 0), (0, C - W_head.shape[1])))
    bhp = jnp.pad(b_head, (0, C - b_head.shape[0]))[None, :]
    out = _head(h_a0[:NSEED], agg_a0p, agg_a1p,
                W_self_a[0], W_neigh_ba[0], b_gnn_a[0][None, :],
                W_self_a[1], W_neigh_ba[1], b_gnn_a[1][None, :],
                Whp, bhp)
    return out[:, :W_head.shape[1]].astype(f32)


# speedup vs baseline: 1.0417x; 1.0417x over previous
import functools

import jax
import jax.numpy as jnp
from jax import lax
from jax.experimental import pallas as pl
from jax.experimental.pallas import tpu as pltpu
from jax.experimental.pallas import tpu_sc as plsc

N = 25000
C = 128
E = 300000
NSEED = 512
NPAD = 25088
EPAD = 307200
HALF = 12544
ACC_BIG = 12560
ACC_SML = 528
ROWBLK = 32
NSLOT = 4
BIGVAL = 1 << 28


def _mesh():
    return plsc.VectorSubcoreMesh(core_axis_name="c", subcore_axis_name="s")


def _compact_loop(srcv, dstv, srcc, dstc, lo, hi, n_vec, fill0=None):
    def body(i, fill):
        vs = srcv[pl.ds(i * 16, 16)]
        vd = dstv[pl.ds(i * 16, 16)]
        vdl = vd - lo
        m = (vd >= lo) & (vd < hi)
        mi = m.astype(jnp.int32)
        pos = fill + jnp.cumsum(mi) - 1
        plsc.store_scatter(srcc, [pos], vs, mask=m)
        plsc.store_scatter(dstc, [pos], vdl, mask=m)
        cnt = jnp.sum(mi, axis=0)
        return fill + cnt
    if fill0 is None:
        fill0 = jnp.int32(0)
    return lax.fori_loop(0, n_vec, body, fill0)


def _pad_tail(srcc, dstc, fill, dummy):
    zs = jnp.zeros((16,), jnp.int32)
    dd = jnp.full((16,), dummy, jnp.int32)
    for j in range(ROWBLK // 16):
        srcc[pl.ds(fill + j * 16, 16)] = zs
        dstc[pl.ds(fill + j * 16, 16)] = dd


def _gather_blk(h_hbm, srcc, rows4, slot, j, sem):
    return pltpu.make_async_copy(
        h_hbm.at[srcc.at[pl.ds(j * ROWBLK, ROWBLK)]], rows4.at[slot], sem)


def _scatter_start(acc, rows4, d2, slot, sem):
    pltpu.async_copy(rows4.at[slot], acc.at[d2.at[slot]], sem, add=True)


def _scatter_wait(acc, rows4, d2, slot, sem):
    pltpu.make_async_copy(rows4.at[slot], acc.at[d2.at[slot]], sem).wait()


def _flush_blocks(h_hbm, acc, srcc, dstc, rows4, d2, gsems, ssems, nb):
    for p in range(NSLOT - 1):
        @pl.when(p < nb)
        def _(p=p):
            _gather_blk(h_hbm, srcc, rows4, p, p, gsems[p]).start()

    def body(j, _):
        def step(slot):
            _gather_blk(h_hbm, srcc, rows4, slot, j, gsems[slot]).wait()
            for t in range(ROWBLK // 16):
                d2[slot, pl.ds(t * 16, 16)] = \
                    dstc[pl.ds(j * ROWBLK + t * 16, 16)]
            _scatter_start(acc, rows4, d2, slot, ssems[slot])

            nslot = (slot + NSLOT - 1) % NSLOT

            @pl.when(j + NSLOT - 1 < nb)
            def _():
                @pl.when(j >= 1)
                def _():
                    _scatter_wait(acc, rows4, d2, nslot, ssems[nslot])
                _gather_blk(h_hbm, srcc, rows4, nslot, j + NSLOT - 1,
                            gsems[nslot]).start()

        for p in range(NSLOT):
            @pl.when(j % NSLOT == p)
            def _(p=p):
                step(p)

        return 0

    lax.fori_loop(0, nb, body, 0)
    for p in range(NSLOT):
        jj = nb - 1 - p

        @pl.when(jj >= 0)
        def _(jj=jj):
            for q in range(NSLOT):
                @pl.when(jj % NSLOT == q)
                def _(q=q):
                    _scatter_wait(acc, rows4, d2, q, ssems[q])


def _zero_acc(acc, zb, s, rows_per_tile):
    z = jnp.zeros((16,), jnp.float32)
    for i in range(16):
        for j in range(8):
            zb[i, pl.ds(j * 16, 16)] = z
    row0 = s * rows_per_tile

    def body(k, _):
        pltpu.sync_copy(zb, acc.at[pl.ds(row0 + k * 16, 16)])
        return 0

    lax.fori_loop(0, rows_per_tile // 16, body, 0)


_EPT_BIG = EPAD // 16
_EPT_SML = EPAD // 32
_CHUNK = 1600
_NCH_BIG = _EPT_BIG // _CHUNK
_NCH_SML = _EPT_SML // _CHUNK


def _seg_phase(h_hbm, src_hbm, dst_hbm, acc, bufs, gsems, ssems, semS,
               e_base, nch, lo, hi, dummy):
    srcv, dstv, srcc, dstc, rows4, d2 = bufs

    def stage(k):
        return (pltpu.make_async_copy(
                    src_hbm.at[pl.ds(e_base + k * _CHUNK, _CHUNK)], srcv, semS),
                pltpu.make_async_copy(
                    dst_hbm.at[pl.ds(e_base + k * _CHUNK, _CHUNK)], dstv, semS))

    for cp in stage(0):
        cp.start()

    def chunk(k, fill):
        for cp in stage(k):
            cp.wait()
        fill = _compact_loop(srcv, dstv, srcc, dstc, lo, hi, _CHUNK // 16, fill)

        @pl.when(k + 1 < nch)
        def _():
            for cp in stage(k + 1):
                cp.start()

        nb = fill // ROWBLK
        _flush_blocks(h_hbm, acc, srcc, dstc, rows4, d2, gsems, ssems, nb)
        base = nb * ROWBLK
        for j in range(ROWBLK // 16):
            vs = srcc[pl.ds(base + j * 16, 16)]
            vd = dstc[pl.ds(base + j * 16, 16)]
            srcc[pl.ds(j * 16, 16)] = vs
            dstc[pl.ds(j * 16, 16)] = vd
        return fill - base

    fill = lax.fori_loop(0, nch, chunk, jnp.int32(0))
    _pad_tail(srcc, dstc, fill, dummy)
    _flush_blocks(h_hbm, acc, srcc, dstc, rows4, d2, gsems, ssems,
                  (fill + (ROWBLK - 1)) // ROWBLK)


@functools.partial(
    pl.kernel,
    out_type=jax.ShapeDtypeStruct((NPAD, C), jnp.float32),
    mesh=_mesh(),
    compiler_params=pltpu.CompilerParams(needs_layout_passes=False),
    scratch_types=[
        pltpu.VMEM_SHARED((ACC_BIG, C), jnp.float32),
        pltpu.VMEM((_CHUNK,), jnp.int32),
        pltpu.VMEM((_CHUNK,), jnp.int32),
        pltpu.VMEM((_CHUNK + ROWBLK,), jnp.int32),
        pltpu.VMEM((_CHUNK + ROWBLK,), jnp.int32),
        pltpu.VMEM((NSLOT, ROWBLK, C), jnp.float32),
        pltpu.VMEM((NSLOT, ROWBLK), jnp.int32),
        pltpu.VMEM((16, C), jnp.float32),
        pltpu.SemaphoreType.DMA,
        pltpu.SemaphoreType.DMA,
        pltpu.SemaphoreType.DMA,
        pltpu.SemaphoreType.DMA,
        pltpu.SemaphoreType.DMA,
        pltpu.SemaphoreType.DMA,
        pltpu.SemaphoreType.DMA,
        pltpu.SemaphoreType.DMA,
        pltpu.SemaphoreType.DMA,
    ],
)
def _segsum_full(ha_hbm, srcab, dstab, out_big,
                 acc, srcv, dstv, srcc, dstc, rows4, d2, zb,
                 g0, g1, g2, g3, s0, s1, s2, s3, semS):
    gsems = [g0, g1, g2, g3]
    ssems = [s0, s1, s2, s3]
    bufs = (srcv, dstv, srcc, dstc, rows4, d2)
    c = lax.axis_index("c")
    s = lax.axis_index("s")
    lo = c * HALF

    _zero_acc(acc, zb, s, HALF // 16)

    @pl.when(s == 0)
    def _():
        pltpu.sync_copy(zb, acc.at[pl.ds(HALF, 16)])

    plsc.subcore_barrier()
    _seg_phase(ha_hbm, srcab, dstab, acc, bufs, gsems, ssems, semS,
               s * _EPT_BIG, _NCH_BIG, lo, lo + HALF, HALF)
    plsc.subcore_barrier()
    per = HALF // 16
    pltpu.sync_copy(acc.at[pl.ds(s * per, per)],
                    out_big.at[pl.ds(c * HALF + s * per, per)])


@functools.partial(
    pl.kernel,
    out_type=jax.ShapeDtypeStruct((2, NSEED, C), jnp.float32),
    mesh=_mesh(),
    compiler_params=pltpu.CompilerParams(needs_layout_passes=False),
    scratch_types=[
        pltpu.VMEM_SHARED((ACC_SML, C), jnp.float32),
        pltpu.VMEM((_EPT_SML,), jnp.int32),
        pltpu.VMEM((_EPT_SML,), jnp.int32),
        pltpu.VMEM((_EPT_SML + ROWBLK,), jnp.int32),
        pltpu.VMEM((_EPT_SML + ROWBLK,), jnp.int32),
        pltpu.VMEM((NSLOT, ROWBLK, C), jnp.float32),
        pltpu.VMEM((NSLOT, ROWBLK), jnp.int32),
        pltpu.VMEM((16, C), jnp.float32),
        pltpu.SemaphoreType.DMA,
        pltpu.SemaphoreType.DMA,
        pltpu.SemaphoreType.DMA,
        pltpu.SemaphoreType.DMA,
        pltpu.SemaphoreType.DMA,
        pltpu.SemaphoreType.DMA,
        pltpu.SemaphoreType.DMA,
        pltpu.SemaphoreType.DMA,
    ],
)
def _segsum_seed(h_hbm, src_hbm, dst_hbm, out_hbm,
                 acc, srcv, dstv, srcc, dstc, rows4, d2, zb,
                 g0, g1, g2, g3, s0, s1, s2, s3):
    gsems = [g0, g1, g2, g3]
    ssems = [s0, s1, s2, s3]
    c = lax.axis_index("c")
    s = lax.axis_index("s")
    e0 = (c * 16 + s) * _EPT_SML
    pltpu.sync_copy(src_hbm.at[pl.ds(e0, _EPT_SML)], srcv)
    pltpu.sync_copy(dst_hbm.at[pl.ds(e0, _EPT_SML)], dstv)
    _zero_acc(acc, zb, s, NSEED // 16)

    @pl.when(s == 0)
    def _():
        pltpu.sync_copy(zb, acc.at[pl.ds(NSEED, 16)])

    plsc.subcore_barrier()
    fill = _compact_loop(srcv, dstv, srcc, dstc, jnp.int32(0), jnp.int32(NSEED),
                         _EPT_SML // 16)
    _pad_tail(srcc, dstc, fill, NSEED)
    _flush_blocks(h_hbm, acc, srcc, dstc, rows4, d2, gsems, ssems,
                  (fill + (ROWBLK - 1)) // ROWBLK)
    plsc.subcore_barrier()
    per = NSEED // 16
    pltpu.sync_copy(acc.at[pl.ds(s * per, per)],
                    out_hbm.at[c, pl.ds(s * per, per)])


_RB = 256


def _enc2_body(xa_ref, ba_ref, ta_ref, xb_ref, bb_ref, tb_ref, seed_ref,
               Wa_ref, bia_ref, Wta_ref, Wb_ref, bib_ref, Wtb_ref,
               oa_ref, ob_ref):
    seed_row = seed_ref[...]
    cols = lax.broadcasted_iota(jnp.int32, (_RB, NSEED), 1)

    def enc(x_ref, b2_ref, t2_ref, W_ref, bias_ref, Wt_ref, o_ref):
        oh = (b2_ref[...] == cols).astype(jnp.float32)
        st = jnp.sum(oh * seed_row, axis=1, keepdims=True)
        dt = st - t2_ref[...]
        h = jnp.dot(x_ref[...], W_ref[...], preferred_element_type=jnp.float32)
        o_ref[...] = h + bias_ref[...] + dt * Wt_ref[...]

    enc(xa_ref, ba_ref, ta_ref, Wa_ref, bia_ref, Wta_ref, oa_ref)
    enc(xb_ref, bb_ref, tb_ref, Wb_ref, bib_ref, Wtb_ref, ob_ref)


def _encode2(xa, ba2, ta2, xb, bb2, tb2, seed_row,
             Wa, bias_a, Wta, Wb, bias_b, Wtb):
    grid = (NPAD // _RB,)
    row = pl.BlockSpec((_RB, C), lambda i: (i, 0))
    one = pl.BlockSpec((_RB, 1), lambda i: (i, 0))
    wspec = pl.BlockSpec((C, C), lambda i: (0, 0))
    vspec = pl.BlockSpec((1, C), lambda i: (0, 0))
    return pl.pallas_call(
        _enc2_body,
        grid=grid,
        in_specs=[row, one, one, row, one, one,
                  pl.BlockSpec((1, NSEED), lambda i: (0, 0)),
                  wspec, vspec, vspec, wspec, vspec, vspec],
        out_specs=[row, row],
        out_shape=[jax.ShapeDtypeStruct((NPAD, C), jnp.float32),
                   jax.ShapeDtypeStruct((NPAD, C), jnp.float32)],
    )(xa, ba2, ta2, xb, bb2, tb2, seed_row,
      Wa, bias_a, Wta, Wb, bias_b, Wtb)


def _combine_body(h_ref, a_ref, Ws_ref, Wn_ref, b_ref, o_ref):
    acc = jnp.dot(h_ref[...], Ws_ref[...], preferred_element_type=jnp.float32)
    acc = acc + jnp.dot(a_ref[...], Wn_ref[...], preferred_element_type=jnp.float32)
    o_ref[...] = jnp.maximum(acc + b_ref[...], 0.0)


def _combine(h, agg, Ws, Wn, b):
    grid = (NPAD // _RB,)
    return pl.pallas_call(
        _combine_body,
        grid=grid,
        in_specs=[
            pl.BlockSpec((_RB, C), lambda i: (i, 0)),
            pl.BlockSpec((_RB, C), lambda i: (i, 0)),
            pl.BlockSpec((C, C), lambda i: (0, 0)),
            pl.BlockSpec((C, C), lambda i: (0, 0)),
            pl.BlockSpec((1, C), lambda i: (0, 0)),
        ],
        out_specs=pl.BlockSpec((_RB, C), lambda i: (i, 0)),
        out_shape=jax.ShapeDtypeStruct((NPAD, C), jnp.float32),
    )(h, agg, Ws, Wn, b)


def _head_body(h_ref, a0_ref, a1_ref, Ws0_ref, Wn0_ref, b0_ref,
               Ws1_ref, Wn1_ref, b1_ref, Wh_ref, bh_ref, o_ref):
    agg0 = a0_ref[0] + a0_ref[1]
    agg1 = a1_ref[0] + a1_ref[1]
    h1 = jnp.dot(h_ref[...], Ws0_ref[...], preferred_element_type=jnp.float32)
    h1 = h1 + jnp.dot(agg0, Wn0_ref[...], preferred_element_type=jnp.float32)
    h1 = jnp.maximum(h1 + b0_ref[...], 0.0)
    h2 = jnp.dot(h1, Ws1_ref[...], preferred_element_type=jnp.float32)
    h2 = h2 + jnp.dot(agg1, Wn1_ref[...], preferred_element_type=jnp.float32)
    h2 = jnp.maximum(h2 + b1_ref[...], 0.0)
    o_ref[...] = jnp.dot(h2, Wh_ref[...], preferred_element_type=jnp.float32) + bh_ref[...]


def _head(h512, a0p, a1p, Ws0, Wn0, b0, Ws1, Wn1, b1, Whp, bhp):
    return pl.pallas_call(
        _head_body,
        grid=(1,),
        in_specs=[
            pl.BlockSpec((NSEED, C), lambda i: (0, 0)),
            pl.BlockSpec((2, NSEED, C), lambda i: (0, 0, 0)),
            pl.BlockSpec((2, NSEED, C), lambda i: (0, 0, 0)),
            pl.BlockSpec((C, C), lambda i: (0, 0)),
            pl.BlockSpec((C, C), lambda i: (0, 0)),
            pl.BlockSpec((1, C), lambda i: (0, 0)),
            pl.BlockSpec((C, C), lambda i: (0, 0)),
            pl.BlockSpec((C, C), lambda i: (0, 0)),
            pl.BlockSpec((1, C), lambda i: (0, 0)),
            pl.BlockSpec((C, C), lambda i: (0, 0)),
            pl.BlockSpec((1, C), lambda i: (0, 0)),
        ],
        out_specs=pl.BlockSpec((NSEED, C), lambda i: (0, 0)),
        out_shape=jax.ShapeDtypeStruct((NSEED, C), jnp.float32),
    )(h512, a0p, a1p, Ws0, Wn0, b0, Ws1, Wn1, b1, Whp, bhp)


def _pad_rows(x, n):
    return jnp.pad(x, ((0, n - x.shape[0]),) + ((0, 0),) * (x.ndim - 1))


def _prep_edges(edge_index):
    src = edge_index[0].astype(jnp.int32)
    dst = edge_index[1].astype(jnp.int32)
    src = jnp.pad(src, (0, EPAD - E))
    dst = jnp.pad(dst, (0, EPAD - E), constant_values=BIGVAL)
    return src, dst


def kernel(x_a, x_b, seed_time, time_a, time_b, edge_index_ab, edge_index_ba,
           batch_a, batch_b, W_enc_a, b_enc_a, W_enc_b, b_enc_b, W_time_a,
           b_time_a, W_time_b, b_time_b, W_self_a, W_neigh_ba, b_gnn_a,
           W_self_b, W_neigh_ab, b_gnn_b, W_head, b_head):
    f32 = jnp.float32
    xa = _pad_rows(x_a, NPAD)
    xb = _pad_rows(x_b, NPAD)
    ba2 = _pad_rows(batch_a.astype(jnp.int32)[:, None], NPAD)
    bb2 = _pad_rows(batch_b.astype(jnp.int32)[:, None], NPAD)
    ta2 = _pad_rows(time_a[:, None], NPAD)
    tb2 = _pad_rows(time_b[:, None], NPAD)
    seed_row = seed_time[None, :]
    src_ab, dst_ab = _prep_edges(edge_index_ab)
    src_ba, dst_ba = _prep_edges(edge_index_ba)

    h_a0, h_b0 = _encode2(xa, ba2, ta2, xb, bb2, tb2, seed_row,
                          W_enc_a, b_enc_a[None, :], W_time_a,
                          W_enc_b, b_enc_b[None, :], W_time_b)

    agg_b0 = _segsum_full(h_a0, src_ab, dst_ab)
    agg_a0p = _segsum_seed(h_b0, src_ba, dst_ba)

    h_b1 = _combine(h_b0, agg_b0, W_self_b[0], W_neigh_ab[0], b_gnn_b[0][None, :])
    agg_a1p = _segsum_seed(h_b1, src_ba, dst_ba)

    Whp = jnp.pad(W_head, ((0, 0), (0, C - W_head.shape[1])))
    bhp = jnp.pad(b_head, (0, C - b_head.shape[0]))[None, :]
    out = _head(h_a0[:NSEED], agg_a0p, agg_a1p,
                W_self_a[0], W_neigh_ba[0], b_gnn_a[0][None, :],
                W_self_a[1], W_neigh_ba[1], b_gnn_a[1][None, :],
                Whp, bhp)
    return out[:, :W_head.shape[1]].astype(f32)


# speedup vs baseline: 1.1789x; 1.1316x over previous
import functools

import jax
import jax.numpy as jnp
from jax import lax
from jax.experimental import pallas as pl
from jax.experimental.pallas import tpu as pltpu
from jax.experimental.pallas import tpu_sc as plsc

N = 25000
C = 128
E = 300000
NSEED = 512
NPAD = 25088
EPAD = 307200
HALF = 12544
ACC_BIG = 12560
ACC_SML = 528
ROWBLK = 32
NSLOT = 4
BIGVAL = 1 << 28


def _mesh():
    return plsc.VectorSubcoreMesh(core_axis_name="c", subcore_axis_name="s")


def _compact_loop(srcv, dstv, srcc, dstc, lo, hi, n_vec, fill0=None):
    def body(i, fill):
        vs = srcv[pl.ds(i * 16, 16)]
        vd = dstv[pl.ds(i * 16, 16)]
        vdl = vd - lo
        m = (vd >= lo) & (vd < hi)
        mi = m.astype(jnp.int32)
        pos = fill + jnp.cumsum(mi) - 1
        plsc.store_scatter(srcc, [pos], vs, mask=m)
        plsc.store_scatter(dstc, [pos], vdl, mask=m)
        cnt = jnp.sum(mi, axis=0)
        return fill + cnt
    if fill0 is None:
        fill0 = jnp.int32(0)
    return lax.fori_loop(0, n_vec, body, fill0)


def _pad_tail(srcc, dstc, fill, dummy):
    zs = jnp.zeros((16,), jnp.int32)
    dd = jnp.full((16,), dummy, jnp.int32)
    for j in range(ROWBLK // 16):
        srcc[pl.ds(fill + j * 16, 16)] = zs
        dstc[pl.ds(fill + j * 16, 16)] = dd


def _gather_blk(h_hbm, srcc, rows4, slot, j, sem):
    return pltpu.make_async_copy(
        h_hbm.at[srcc.at[pl.ds(j * ROWBLK, ROWBLK)]], rows4.at[slot], sem)


def _scatter_start(acc, rows4, d2, slot, sem):
    pltpu.async_copy(rows4.at[slot], acc.at[d2.at[slot]], sem, add=True)


def _scatter_wait(acc, rows4, d2, slot, sem):
    pltpu.make_async_copy(rows4.at[slot], acc.at[d2.at[slot]], sem).wait()


def _flush_blocks(h_hbm, acc, srcc, dstc, rows4, d2, gsems, ssems, nb):
    for p in range(NSLOT - 1):
        @pl.when(p < nb)
        def _(p=p):
            _gather_blk(h_hbm, srcc, rows4, p, p, gsems[p]).start()

    def body(j, _):
        def step(slot):
            _gather_blk(h_hbm, srcc, rows4, slot, j, gsems[slot]).wait()
            for t in range(ROWBLK // 16):
                d2[slot, pl.ds(t * 16, 16)] = \
                    dstc[pl.ds(j * ROWBLK + t * 16, 16)]
            _scatter_start(acc, rows4, d2, slot, ssems[slot])

            nslot = (slot + NSLOT - 1) % NSLOT

            @pl.when(j + NSLOT - 1 < nb)
            def _():
                @pl.when(j >= 1)
                def _():
                    _scatter_wait(acc, rows4, d2, nslot, ssems[nslot])
                _gather_blk(h_hbm, srcc, rows4, nslot, j + NSLOT - 1,
                            gsems[nslot]).start()

        for p in range(NSLOT):
            @pl.when(j % NSLOT == p)
            def _(p=p):
                step(p)

        return 0

    lax.fori_loop(0, nb, body, 0)
    for p in range(NSLOT):
        jj = nb - 1 - p

        @pl.when(jj >= 0)
        def _(jj=jj):
            for q in range(NSLOT):
                @pl.when(jj % NSLOT == q)
                def _(q=q):
                    _scatter_wait(acc, rows4, d2, q, ssems[q])


def _zero_acc(acc, zb, s, rows_per_tile):
    z = jnp.zeros((16,), jnp.float32)
    for i in range(16):
        for j in range(8):
            zb[i, pl.ds(j * 16, 16)] = z
    row0 = s * rows_per_tile

    def body(k, _):
        pltpu.sync_copy(zb, acc.at[pl.ds(row0 + k * 16, 16)])
        return 0

    lax.fori_loop(0, rows_per_tile // 16, body, 0)


_EPT_BIG = EPAD // 16
_EPT_SML = EPAD // 32
_CHUNK = 1600
_NCH_BIG = _EPT_BIG // _CHUNK
_NCH_SML = _EPT_SML // _CHUNK


def _seg_phase(h_hbm, src_hbm, dst_hbm, acc, bufs, gsems, ssems, semS,
               e_base, nch, lo, hi, dummy):
    srcv, dstv, srcc, dstc, rows4, d2 = bufs

    def stage(k):
        return (pltpu.make_async_copy(
                    src_hbm.at[pl.ds(e_base + k * _CHUNK, _CHUNK)], srcv, semS),
                pltpu.make_async_copy(
                    dst_hbm.at[pl.ds(e_base + k * _CHUNK, _CHUNK)], dstv, semS))

    for cp in stage(0):
        cp.start()

    def chunk(k, fill):
        for cp in stage(k):
            cp.wait()
        fill = _compact_loop(srcv, dstv, srcc, dstc, lo, hi, _CHUNK // 16, fill)

        @pl.when(k + 1 < nch)
        def _():
            for cp in stage(k + 1):
                cp.start()

        nb = fill // ROWBLK
        _flush_blocks(h_hbm, acc, srcc, dstc, rows4, d2, gsems, ssems, nb)
        base = nb * ROWBLK
        for j in range(ROWBLK // 16):
            vs = srcc[pl.ds(base + j * 16, 16)]
            vd = dstc[pl.ds(base + j * 16, 16)]
            srcc[pl.ds(j * 16, 16)] = vs
            dstc[pl.ds(j * 16, 16)] = vd
        return fill - base

    fill = lax.fori_loop(0, nch, chunk, jnp.int32(0))
    _pad_tail(srcc, dstc, fill, dummy)
    _flush_blocks(h_hbm, acc, srcc, dstc, rows4, d2, gsems, ssems,
                  (fill + (ROWBLK - 1)) // ROWBLK)


@functools.partial(
    pl.kernel,
    out_type=jax.ShapeDtypeStruct((NPAD, C), jnp.float32),
    mesh=_mesh(),
    compiler_params=pltpu.CompilerParams(needs_layout_passes=False),
    scratch_types=[
        pltpu.VMEM_SHARED((ACC_BIG, C), jnp.float32),
        pltpu.VMEM((_CHUNK,), jnp.int32),
        pltpu.VMEM((_CHUNK,), jnp.int32),
        pltpu.VMEM((_CHUNK + ROWBLK,), jnp.int32),
        pltpu.VMEM((_CHUNK + ROWBLK,), jnp.int32),
        pltpu.VMEM((NSLOT, ROWBLK, C), jnp.float32),
        pltpu.VMEM((NSLOT, ROWBLK), jnp.int32),
        pltpu.VMEM((16, C), jnp.float32),
        pltpu.SemaphoreType.DMA,
        pltpu.SemaphoreType.DMA,
        pltpu.SemaphoreType.DMA,
        pltpu.SemaphoreType.DMA,
        pltpu.SemaphoreType.DMA,
        pltpu.SemaphoreType.DMA,
        pltpu.SemaphoreType.DMA,
        pltpu.SemaphoreType.DMA,
        pltpu.SemaphoreType.DMA,
    ],
)
def _segsum_full(ha_hbm, srcab, dstab, out_big,
                 acc, srcv, dstv, srcc, dstc, rows4, d2, zb,
                 g0, g1, g2, g3, s0, s1, s2, s3, semS):
    gsems = [g0, g1, g2, g3]
    ssems = [s0, s1, s2, s3]
    bufs = (srcv, dstv, srcc, dstc, rows4, d2)
    c = lax.axis_index("c")
    s = lax.axis_index("s")
    lo = c * HALF

    _zero_acc(acc, zb, s, HALF // 16)

    @pl.when(s == 0)
    def _():
        pltpu.sync_copy(zb, acc.at[pl.ds(HALF, 16)])

    plsc.subcore_barrier()
    _seg_phase(ha_hbm, srcab, dstab, acc, bufs, gsems, ssems, semS,
               s * _EPT_BIG, _NCH_BIG, lo, lo + HALF, HALF)
    plsc.subcore_barrier()
    per = HALF // 16
    pltpu.sync_copy(acc.at[pl.ds(s * per, per)],
                    out_big.at[pl.ds(c * HALF + s * per, per)])


@functools.partial(
    pl.kernel,
    out_type=jax.ShapeDtypeStruct((2, NSEED, C), jnp.float32),
    mesh=_mesh(),
    compiler_params=pltpu.CompilerParams(needs_layout_passes=False),
    scratch_types=[
        pltpu.VMEM_SHARED((ACC_SML, C), jnp.float32),
        pltpu.VMEM((_EPT_SML,), jnp.int32),
        pltpu.VMEM((_EPT_SML,), jnp.int32),
        pltpu.VMEM((_EPT_SML + ROWBLK,), jnp.int32),
        pltpu.VMEM((_EPT_SML + ROWBLK,), jnp.int32),
        pltpu.VMEM((NSLOT, ROWBLK, C), jnp.float32),
        pltpu.VMEM((NSLOT, ROWBLK), jnp.int32),
        pltpu.VMEM((16, C), jnp.float32),
        pltpu.SemaphoreType.DMA,
        pltpu.SemaphoreType.DMA,
        pltpu.SemaphoreType.DMA,
        pltpu.SemaphoreType.DMA,
        pltpu.SemaphoreType.DMA,
        pltpu.SemaphoreType.DMA,
        pltpu.SemaphoreType.DMA,
        pltpu.SemaphoreType.DMA,
    ],
)
def _segsum_seed(h_hbm, src_hbm, dst_hbm, out_hbm,
                 acc, srcv, dstv, srcc, dstc, rows4, d2, zb,
                 g0, g1, g2, g3, s0, s1, s2, s3):
    gsems = [g0, g1, g2, g3]
    ssems = [s0, s1, s2, s3]
    c = lax.axis_index("c")
    s = lax.axis_index("s")
    e0 = (c * 16 + s) * _EPT_SML
    pltpu.sync_copy(src_hbm.at[pl.ds(e0, _EPT_SML)], srcv)
    pltpu.sync_copy(dst_hbm.at[pl.ds(e0, _EPT_SML)], dstv)
    _zero_acc(acc, zb, s, NSEED // 16)

    @pl.when(s == 0)
    def _():
        pltpu.sync_copy(zb, acc.at[pl.ds(NSEED, 16)])

    plsc.subcore_barrier()
    fill = _compact_loop(srcv, dstv, srcc, dstc, jnp.int32(0), jnp.int32(NSEED),
                         _EPT_SML // 16)
    _pad_tail(srcc, dstc, fill, NSEED)
    _flush_blocks(h_hbm, acc, srcc, dstc, rows4, d2, gsems, ssems,
                  (fill + (ROWBLK - 1)) // ROWBLK)
    plsc.subcore_barrier()
    per = NSEED // 16
    pltpu.sync_copy(acc.at[pl.ds(s * per, per)],
                    out_hbm.at[c, pl.ds(s * per, per)])


_RB = 256


def _enc_body(x_ref, b2_ref, t2_ref, seed_ref, W_ref, bias_ref, Wt_ref, o_ref):
    x = x_ref[...]
    batch = b2_ref[...]
    tv = t2_ref[...]
    seed_row = seed_ref[...]
    cols = lax.broadcasted_iota(jnp.int32, (_RB, NSEED), 1)
    oh = (batch == cols).astype(jnp.float32)
    st = jnp.sum(oh * seed_row, axis=1, keepdims=True)
    dt = st - tv
    h = jnp.dot(x, W_ref[...], preferred_element_type=jnp.float32)
    h = h + bias_ref[...] + dt * Wt_ref[...]
    o_ref[...] = h


def _encode(x, b2, t2, seed_row, W, bias, Wt):
    grid = (NPAD // _RB,)
    return pl.pallas_call(
        _enc_body,
        grid=grid,
        in_specs=[
            pl.BlockSpec((_RB, C), lambda i: (i, 0)),
            pl.BlockSpec((_RB, 1), lambda i: (i, 0)),
            pl.BlockSpec((_RB, 1), lambda i: (i, 0)),
            pl.BlockSpec((1, NSEED), lambda i: (0, 0)),
            pl.BlockSpec((C, C), lambda i: (0, 0)),
            pl.BlockSpec((1, C), lambda i: (0, 0)),
            pl.BlockSpec((1, C), lambda i: (0, 0)),
        ],
        out_specs=pl.BlockSpec((_RB, C), lambda i: (i, 0)),
        out_shape=jax.ShapeDtypeStruct((NPAD, C), jnp.float32),
    )(x, b2, t2, seed_row, W, bias, Wt)


def _combine_body(h_ref, a_ref, Ws_ref, Wn_ref, b_ref, o_ref):
    acc = jnp.dot(h_ref[...], Ws_ref[...], preferred_element_type=jnp.float32)
    acc = acc + jnp.dot(a_ref[...], Wn_ref[...], preferred_element_type=jnp.float32)
    o_ref[...] = jnp.maximum(acc + b_ref[...], 0.0)


def _combine(h, agg, Ws, Wn, b):
    grid = (NPAD // _RB,)
    return pl.pallas_call(
        _combine_body,
        grid=grid,
        in_specs=[
            pl.BlockSpec((_RB, C), lambda i: (i, 0)),
            pl.BlockSpec((_RB, C), lambda i: (i, 0)),
            pl.BlockSpec((C, C), lambda i: (0, 0)),
            pl.BlockSpec((C, C), lambda i: (0, 0)),
            pl.BlockSpec((1, C), lambda i: (0, 0)),
        ],
        out_specs=pl.BlockSpec((_RB, C), lambda i: (i, 0)),
        out_shape=jax.ShapeDtypeStruct((NPAD, C), jnp.float32),
    )(h, agg, Ws, Wn, b)


def _head_body(h_ref, a0_ref, a1_ref, Ws0_ref, Wn0_ref, b0_ref,
               Ws1_ref, Wn1_ref, b1_ref, Wh_ref, bh_ref, o_ref):
    agg0 = a0_ref[0] + a0_ref[1]
    agg1 = a1_ref[0] + a1_ref[1]
    h1 = jnp.dot(h_ref[...], Ws0_ref[...], preferred_element_type=jnp.float32)
    h1 = h1 + jnp.dot(agg0, Wn0_ref[...], preferred_element_type=jnp.float32)
    h1 = jnp.maximum(h1 + b0_ref[...], 0.0)
    h2 = jnp.dot(h1, Ws1_ref[...], preferred_element_type=jnp.float32)
    h2 = h2 + jnp.dot(agg1, Wn1_ref[...], preferred_element_type=jnp.float32)
    h2 = jnp.maximum(h2 + b1_ref[...], 0.0)
    o_ref[...] = jnp.dot(h2, Wh_ref[...], preferred_element_type=jnp.float32) + bh_ref[...]


def _head(h512, a0p, a1p, Ws0, Wn0, b0, Ws1, Wn1, b1, Whp, bhp):
    return pl.pallas_call(
        _head_body,
        grid=(1,),
        in_specs=[
            pl.BlockSpec((NSEED, C), lambda i: (0, 0)),
            pl.BlockSpec((2, NSEED, C), lambda i: (0, 0, 0)),
            pl.BlockSpec((2, NSEED, C), lambda i: (0, 0, 0)),
            pl.BlockSpec((C, C), lambda i: (0, 0)),
            pl.BlockSpec((C, C), lambda i: (0, 0)),
            pl.BlockSpec((1, C), lambda i: (0, 0)),
            pl.BlockSpec((C, C), lambda i: (0, 0)),
            pl.BlockSpec((C, C), lambda i: (0, 0)),
            pl.BlockSpec((1, C), lambda i: (0, 0)),
            pl.BlockSpec((C, C), lambda i: (0, 0)),
            pl.BlockSpec((1, C), lambda i: (0, 0)),
        ],
        out_specs=pl.BlockSpec((NSEED, C), lambda i: (0, 0)),
        out_shape=jax.ShapeDtypeStruct((NSEED, C), jnp.float32),
    )(h512, a0p, a1p, Ws0, Wn0, b0, Ws1, Wn1, b1, Whp, bhp)


def _pad_rows(x, n):
    return jnp.pad(x, ((0, n - x.shape[0]),) + ((0, 0),) * (x.ndim - 1))


def _prep_edges(edge_index):
    src = edge_index[0].astype(jnp.int32)
    dst = edge_index[1].astype(jnp.int32)
    src = jnp.pad(src, (0, EPAD - E))
    dst = jnp.pad(dst, (0, EPAD - E), constant_values=BIGVAL)
    return src, dst


def kernel(x_a, x_b, seed_time, time_a, time_b, edge_index_ab, edge_index_ba,
           batch_a, batch_b, W_enc_a, b_enc_a, W_enc_b, b_enc_b, W_time_a,
           b_time_a, W_time_b, b_time_b, W_self_a, W_neigh_ba, b_gnn_a,
           W_self_b, W_neigh_ab, b_gnn_b, W_head, b_head):
    f32 = jnp.float32
    xa = _pad_rows(x_a, NPAD)
    xb = _pad_rows(x_b, NPAD)
    ba2 = _pad_rows(batch_a.astype(jnp.int32)[:, None], NPAD)
    bb2 = _pad_rows(batch_b.astype(jnp.int32)[:, None], NPAD)
    ta2 = _pad_rows(time_a[:, None], NPAD)
    tb2 = _pad_rows(time_b[:, None], NPAD)
    seed_row = seed_time[None, :]
    src_ab, dst_ab = _prep_edges(edge_index_ab)
    src_ba, dst_ba = _prep_edges(edge_index_ba)

    h_a0 = _encode(xa, ba2, ta2, seed_row, W_enc_a, b_enc_a[None, :], W_time_a)
    h_b0 = _encode(xb, bb2, tb2, seed_row, W_enc_b, b_enc_b[None, :], W_time_b)

    agg_b0 = _segsum_full(h_a0, src_ab, dst_ab)
    agg_a0p = _segsum_seed(h_b0, src_ba, dst_ba)

    h_b1 = _combine(h_b0, agg_b0, W_self_b[0], W_neigh_ab[0], b_gnn_b[0][None, :])
    agg_a1p = _segsum_seed(h_b1, src_ba, dst_ba)

    Whp = jnp.pad(W_head, ((0, 0), (0, C - W_head.shape[1])))
    bhp = jnp.pad(b_head, (0, C - b_head.shape[0]))[None, :]
    out = _head(h_a0[:NSEED], agg_a0p, agg_a1p,
                W_self_a[0], W_neigh_ba[0], b_gnn_a[0][None, :],
                W_self_a[1], W_neigh_ba[1], b_gnn_a[1][None, :],
                Whp, bhp)
    return out[:, :W_head.shape[1]].astype(f32)


# speedup vs baseline: 1.2060x; 1.0230x over previous
import functools

import jax
import jax.numpy as jnp
from jax import lax
from jax.experimental import pallas as pl
from jax.experimental.pallas import tpu as pltpu
from jax.experimental.pallas import tpu_sc as plsc

N = 25000
C = 128
E = 300000
NSEED = 512
NPAD = 25088
EPAD = 307200
HALF = 12544
ACC_BIG = 12560
ACC_SML = 528
ROWBLK = 32
NSLOT = 5
BIGVAL = 1 << 28


def _mesh():
    return plsc.VectorSubcoreMesh(core_axis_name="c", subcore_axis_name="s")


def _compact_loop(srcv, dstv, srcc, dstc, lo, hi, n_vec, fill0=None):
    def body(i, fill):
        vs = srcv[pl.ds(i * 16, 16)]
        vd = dstv[pl.ds(i * 16, 16)]
        vdl = vd - lo
        m = (vd >= lo) & (vd < hi)
        mi = m.astype(jnp.int32)
        pos = fill + jnp.cumsum(mi) - 1
        plsc.store_scatter(srcc, [pos], vs, mask=m)
        plsc.store_scatter(dstc, [pos], vdl, mask=m)
        cnt = jnp.sum(mi, axis=0)
        return fill + cnt
    if fill0 is None:
        fill0 = jnp.int32(0)
    return lax.fori_loop(0, n_vec, body, fill0)


def _pad_tail(srcc, dstc, fill, dummy):
    zs = jnp.zeros((16,), jnp.int32)
    dd = jnp.full((16,), dummy, jnp.int32)
    for j in range(ROWBLK // 16):
        srcc[pl.ds(fill + j * 16, 16)] = zs
        dstc[pl.ds(fill + j * 16, 16)] = dd


def _gather_blk(h_hbm, srcc, rows4, slot, j, sem):
    return pltpu.make_async_copy(
        h_hbm.at[srcc.at[pl.ds(j * ROWBLK, ROWBLK)]], rows4.at[slot], sem)


def _scatter_start(acc, dstc, rows4, slot, j, sem):
    pltpu.async_copy(rows4.at[slot],
                     acc.at[dstc.at[pl.ds(j * ROWBLK, ROWBLK)]], sem, add=True)


def _scatter_wait(acc, dstc, rows4, slot, j, sem):
    pltpu.make_async_copy(
        rows4.at[slot], acc.at[dstc.at[pl.ds(j * ROWBLK, ROWBLK)]], sem).wait()


def _flush_blocks(h_hbm, acc, srcc, dstc, rows4, gsems, ssems, nb):
    for p in range(NSLOT - 1):
        @pl.when(p < nb)
        def _(p=p):
            _gather_blk(h_hbm, srcc, rows4, p, p, gsems[p]).start()

    def body(j, _):
        def step(slot):
            _gather_blk(h_hbm, srcc, rows4, slot, j, gsems[slot]).wait()
            _scatter_start(acc, dstc, rows4, slot, j, ssems[slot])

            nslot = (slot + NSLOT - 1) % NSLOT

            @pl.when(j + NSLOT - 1 < nb)
            def _():
                @pl.when(j >= 1)
                def _():
                    _scatter_wait(acc, dstc, rows4, nslot, j - 1, ssems[nslot])
                _gather_blk(h_hbm, srcc, rows4, nslot, j + NSLOT - 1,
                            gsems[nslot]).start()

        for p in range(NSLOT):
            @pl.when(j % NSLOT == p)
            def _(p=p):
                step(p)

        return 0

    lax.fori_loop(0, nb, body, 0)
    for p in range(NSLOT):
        jj = nb - 1 - p

        @pl.when(jj >= 0)
        def _(jj=jj):
            for q in range(NSLOT):
                @pl.when(jj % NSLOT == q)
                def _(q=q):
                    _scatter_wait(acc, dstc, rows4, q, jj, ssems[q])


def _zero_acc(acc, zb, s, rows_per_tile):
    z = jnp.zeros((16,), jnp.float32)
    for i in range(16):
        for j in range(8):
            zb[i, pl.ds(j * 16, 16)] = z
    row0 = s * rows_per_tile

    def body(k, _):
        pltpu.sync_copy(zb, acc.at[pl.ds(row0 + k * 16, 16)])
        return 0

    lax.fori_loop(0, rows_per_tile // 16, body, 0)


_EPT_BIG = EPAD // 16
_EPT_SML = EPAD // 32
_CHUNK = 1600
_NCH_BIG = _EPT_BIG // _CHUNK
_NCH_SML = _EPT_SML // _CHUNK


def _seg_phase(h_hbm, src_hbm, dst_hbm, acc, bufs, gsems, ssems, semS,
               e_base, nch, lo, hi, dummy):
    srcv, dstv, srcc, dstc, rows4 = bufs

    def stage(k):
        return (pltpu.make_async_copy(
                    src_hbm.at[pl.ds(e_base + k * _CHUNK, _CHUNK)], srcv, semS),
                pltpu.make_async_copy(
                    dst_hbm.at[pl.ds(e_base + k * _CHUNK, _CHUNK)], dstv, semS))

    for cp in stage(0):
        cp.start()

    def chunk(k, fill):
        for cp in stage(k):
            cp.wait()
        fill = _compact_loop(srcv, dstv, srcc, dstc, lo, hi, _CHUNK // 16, fill)

        @pl.when(k + 1 < nch)
        def _():
            for cp in stage(k + 1):
                cp.start()

        nb = fill // ROWBLK
        _flush_blocks(h_hbm, acc, srcc, dstc, rows4, gsems, ssems, nb)
        base = nb * ROWBLK
        for j in range(ROWBLK // 16):
            vs = srcc[pl.ds(base + j * 16, 16)]
            vd = dstc[pl.ds(base + j * 16, 16)]
            srcc[pl.ds(j * 16, 16)] = vs
            dstc[pl.ds(j * 16, 16)] = vd
        return fill - base

    fill = lax.fori_loop(0, nch, chunk, jnp.int32(0))
    _pad_tail(srcc, dstc, fill, dummy)
    _flush_blocks(h_hbm, acc, srcc, dstc, rows4, gsems, ssems,
                  (fill + (ROWBLK - 1)) // ROWBLK)


@functools.partial(
    pl.kernel,
    out_type=jax.ShapeDtypeStruct((NPAD, C), jnp.float32),
    mesh=_mesh(),
    compiler_params=pltpu.CompilerParams(needs_layout_passes=False),
    scratch_types=[
        pltpu.VMEM_SHARED((ACC_BIG, C), jnp.float32),
        pltpu.VMEM((_CHUNK,), jnp.int32),
        pltpu.VMEM((_CHUNK,), jnp.int32),
        pltpu.VMEM((_CHUNK + ROWBLK,), jnp.int32),
        pltpu.VMEM((_CHUNK + ROWBLK,), jnp.int32),
        pltpu.VMEM((NSLOT, ROWBLK, C), jnp.float32),
        pltpu.VMEM((16, C), jnp.float32),
        pltpu.SemaphoreType.DMA,
        pltpu.SemaphoreType.DMA,
        pltpu.SemaphoreType.DMA,
        pltpu.SemaphoreType.DMA,
        pltpu.SemaphoreType.DMA,
        pltpu.SemaphoreType.DMA,
        pltpu.SemaphoreType.DMA,
        pltpu.SemaphoreType.DMA,
        pltpu.SemaphoreType.DMA,
        pltpu.SemaphoreType.DMA,
        pltpu.SemaphoreType.DMA,
    ],
)
def _segsum_full(ha_hbm, srcab, dstab, out_big,
                 acc, srcv, dstv, srcc, dstc, rows4, zb,
                 g0, g1, g2, g3, g4, s0, s1, s2, s3, s4, semS):
    gsems = [g0, g1, g2, g3, g4]
    ssems = [s0, s1, s2, s3, s4]
    bufs = (srcv, dstv, srcc, dstc, rows4)
    c = lax.axis_index("c")
    s = lax.axis_index("s")
    lo = c * HALF

    _zero_acc(acc, zb, s, HALF // 16)

    @pl.when(s == 0)
    def _():
        pltpu.sync_copy(zb, acc.at[pl.ds(HALF, 16)])

    plsc.subcore_barrier()
    _seg_phase(ha_hbm, srcab, dstab, acc, bufs, gsems, ssems, semS,
               s * _EPT_BIG, _NCH_BIG, lo, lo + HALF, HALF)
    plsc.subcore_barrier()
    per = HALF // 16
    pltpu.sync_copy(acc.at[pl.ds(s * per, per)],
                    out_big.at[pl.ds(c * HALF + s * per, per)])


@functools.partial(
    pl.kernel,
    out_type=jax.ShapeDtypeStruct((2, NSEED, C), jnp.float32),
    mesh=_mesh(),
    compiler_params=pltpu.CompilerParams(needs_layout_passes=False),
    scratch_types=[
        pltpu.VMEM_SHARED((ACC_SML, C), jnp.float32),
        pltpu.VMEM((_EPT_SML,), jnp.int32),
        pltpu.VMEM((_EPT_SML,), jnp.int32),
        pltpu.VMEM((_EPT_SML + ROWBLK,), jnp.int32),
        pltpu.VMEM((_EPT_SML + ROWBLK,), jnp.int32),
        pltpu.VMEM((NSLOT, ROWBLK, C), jnp.float32),
        pltpu.VMEM((16, C), jnp.float32),
        pltpu.SemaphoreType.DMA,
        pltpu.SemaphoreType.DMA,
        pltpu.SemaphoreType.DMA,
        pltpu.SemaphoreType.DMA,
        pltpu.SemaphoreType.DMA,
        pltpu.SemaphoreType.DMA,
        pltpu.SemaphoreType.DMA,
        pltpu.SemaphoreType.DMA,
        pltpu.SemaphoreType.DMA,
        pltpu.SemaphoreType.DMA,
    ],
)
def _segsum_seed(h_hbm, src_hbm, dst_hbm, out_hbm,
                 acc, srcv, dstv, srcc, dstc, rows4, zb,
                 g0, g1, g2, g3, g4, s0, s1, s2, s3, s4):
    gsems = [g0, g1, g2, g3, g4]
    ssems = [s0, s1, s2, s3, s4]
    c = lax.axis_index("c")
    s = lax.axis_index("s")
    e0 = (c * 16 + s) * _EPT_SML
    pltpu.sync_copy(src_hbm.at[pl.ds(e0, _EPT_SML)], srcv)
    pltpu.sync_copy(dst_hbm.at[pl.ds(e0, _EPT_SML)], dstv)
    _zero_acc(acc, zb, s, NSEED // 16)

    @pl.when(s == 0)
    def _():
        pltpu.sync_copy(zb, acc.at[pl.ds(NSEED, 16)])

    plsc.subcore_barrier()
    fill = _compact_loop(srcv, dstv, srcc, dstc, jnp.int32(0), jnp.int32(NSEED),
                         _EPT_SML // 16)
    _pad_tail(srcc, dstc, fill, NSEED)
    _flush_blocks(h_hbm, acc, srcc, dstc, rows4, gsems, ssems,
                  (fill + (ROWBLK - 1)) // ROWBLK)
    plsc.subcore_barrier()
    per = NSEED // 16
    pltpu.sync_copy(acc.at[pl.ds(s * per, per)],
                    out_hbm.at[c, pl.ds(s * per, per)])


_RB = 256


def _enc_body(x_ref, b2_ref, t2_ref, seed_ref, W_ref, bias_ref, Wt_ref, o_ref):
    x = x_ref[...]
    batch = b2_ref[...]
    tv = t2_ref[...]
    seed_row = seed_ref[...]
    cols = lax.broadcasted_iota(jnp.int32, (_RB, NSEED), 1)
    oh = (batch == cols).astype(jnp.float32)
    st = jnp.sum(oh * seed_row, axis=1, keepdims=True)
    dt = st - tv
    h = jnp.dot(x, W_ref[...], preferred_element_type=jnp.float32)
    h = h + bias_ref[...] + dt * Wt_ref[...]
    o_ref[...] = h


def _encode(x, b2, t2, seed_row, W, bias, Wt):
    grid = (NPAD // _RB,)
    return pl.pallas_call(
        _enc_body,
        grid=grid,
        in_specs=[
            pl.BlockSpec((_RB, C), lambda i: (i, 0)),
            pl.BlockSpec((_RB, 1), lambda i: (i, 0)),
            pl.BlockSpec((_RB, 1), lambda i: (i, 0)),
            pl.BlockSpec((1, NSEED), lambda i: (0, 0)),
            pl.BlockSpec((C, C), lambda i: (0, 0)),
            pl.BlockSpec((1, C), lambda i: (0, 0)),
            pl.BlockSpec((1, C), lambda i: (0, 0)),
        ],
        out_specs=pl.BlockSpec((_RB, C), lambda i: (i, 0)),
        out_shape=jax.ShapeDtypeStruct((NPAD, C), jnp.float32),
    )(x, b2, t2, seed_row, W, bias, Wt)


def _combine_body(h_ref, a_ref, Ws_ref, Wn_ref, b_ref, o_ref):
    acc = jnp.dot(h_ref[...], Ws_ref[...], preferred_element_type=jnp.float32)
    acc = acc + jnp.dot(a_ref[...], Wn_ref[...], preferred_element_type=jnp.float32)
    o_ref[...] = jnp.maximum(acc + b_ref[...], 0.0)


def _combine(h, agg, Ws, Wn, b):
    grid = (NPAD // _RB,)
    return pl.pallas_call(
        _combine_body,
        grid=grid,
        in_specs=[
            pl.BlockSpec((_RB, C), lambda i: (i, 0)),
            pl.BlockSpec((_RB, C), lambda i: (i, 0)),
            pl.BlockSpec((C, C), lambda i: (0, 0)),
            pl.BlockSpec((C, C), lambda i: (0, 0)),
            pl.BlockSpec((1, C), lambda i: (0, 0)),
        ],
        out_specs=pl.BlockSpec((_RB, C), lambda i: (i, 0)),
        out_shape=jax.ShapeDtypeStruct((NPAD, C), jnp.float32),
    )(h, agg, Ws, Wn, b)


def _head_body(h_ref, a0_ref, a1_ref, Ws0_ref, Wn0_ref, b0_ref,
               Ws1_ref, Wn1_ref, b1_ref, Wh_ref, bh_ref, o_ref):
    agg0 = a0_ref[0] + a0_ref[1]
    agg1 = a1_ref[0] + a1_ref[1]
    h1 = jnp.dot(h_ref[...], Ws0_ref[...], preferred_element_type=jnp.float32)
    h1 = h1 + jnp.dot(agg0, Wn0_ref[...], preferred_element_type=jnp.float32)
    h1 = jnp.maximum(h1 + b0_ref[...], 0.0)
    h2 = jnp.dot(h1, Ws1_ref[...], preferred_element_type=jnp.float32)
    h2 = h2 + jnp.dot(agg1, Wn1_ref[...], preferred_element_type=jnp.float32)
    h2 = jnp.maximum(h2 + b1_ref[...], 0.0)
    o_ref[...] = jnp.dot(h2, Wh_ref[...], preferred_element_type=jnp.float32) + bh_ref[...]


def _head(h512, a0p, a1p, Ws0, Wn0, b0, Ws1, Wn1, b1, Whp, bhp):
    return pl.pallas_call(
        _head_body,
        grid=(1,),
        in_specs=[
            pl.BlockSpec((NSEED, C), lambda i: (0, 0)),
            pl.BlockSpec((2, NSEED, C), lambda i: (0, 0, 0)),
            pl.BlockSpec((2, NSEED, C), lambda i: (0, 0, 0)),
            pl.BlockSpec((C, C), lambda i: (0, 0)),
            pl.BlockSpec((C, C), lambda i: (0, 0)),
            pl.BlockSpec((1, C), lambda i: (0, 0)),
            pl.BlockSpec((C, C), lambda i: (0, 0)),
            pl.BlockSpec((C, C), lambda i: (0, 0)),
            pl.BlockSpec((1, C), lambda i: (0, 0)),
            pl.BlockSpec((C, C), lambda i: (0, 0)),
            pl.BlockSpec((1, C), lambda i: (0, 0)),
        ],
        out_specs=pl.BlockSpec((NSEED, C), lambda i: (0, 0)),
        out_shape=jax.ShapeDtypeStruct((NSEED, C), jnp.float32),
    )(h512, a0p, a1p, Ws0, Wn0, b0, Ws1, Wn1, b1, Whp, bhp)


def _pad_rows(x, n):
    return jnp.pad(x, ((0, n - x.shape[0]),) + ((0, 0),) * (x.ndim - 1))


def _prep_edges(edge_index):
    src = edge_index[0].astype(jnp.int32)
    dst = edge_index[1].astype(jnp.int32)
    src = jnp.pad(src, (0, EPAD - E))
    dst = jnp.pad(dst, (0, EPAD - E), constant_values=BIGVAL)
    return src, dst


def kernel(x_a, x_b, seed_time, time_a, time_b, edge_index_ab, edge_index_ba,
           batch_a, batch_b, W_enc_a, b_enc_a, W_enc_b, b_enc_b, W_time_a,
           b_time_a, W_time_b, b_time_b, W_self_a, W_neigh_ba, b_gnn_a,
           W_self_b, W_neigh_ab, b_gnn_b, W_head, b_head):
    f32 = jnp.float32
    xa = _pad_rows(x_a, NPAD)
    xb = _pad_rows(x_b, NPAD)
    ba2 = _pad_rows(batch_a.astype(jnp.int32)[:, None], NPAD)
    bb2 = _pad_rows(batch_b.astype(jnp.int32)[:, None], NPAD)
    ta2 = _pad_rows(time_a[:, None], NPAD)
    tb2 = _pad_rows(time_b[:, None], NPAD)
    seed_row = seed_time[None, :]
    src_ab, dst_ab = _prep_edges(edge_index_ab)
    src_ba, dst_ba = _prep_edges(edge_index_ba)

    h_a0 = _encode(xa, ba2, ta2, seed_row, W_enc_a, b_enc_a[None, :], W_time_a)
    h_b0 = _encode(xb, bb2, tb2, seed_row, W_enc_b, b_enc_b[None, :], W_time_b)

    agg_b0 = _segsum_full(h_a0, src_ab, dst_ab)
    agg_a0p = _segsum_seed(h_b0, src_ba, dst_ba)

    h_b1 = _combine(h_b0, agg_b0, W_self_b[0], W_neigh_ab[0], b_gnn_b[0][None, :])
    agg_a1p = _segsum_seed(h_b1, src_ba, dst_ba)

    Whp = jnp.pad(W_head, ((0, 0), (0, C - W_head.shape[1])))
    bhp = jnp.pad(b_head, (0, C - b_head.shape[0]))[None, :]
    out = _head(h_a0[:NSEED], agg_a0p, agg_a1p,
                W_self_a[0], W_neigh_ba[0], b_gnn_a[0][None, :],
                W_self_a[1], W_neigh_ba[1], b_gnn_a[1][None, :],
                Whp, bhp)
    return out[:, :W_head.shape[1]].astype(f32)
